# baseline (speedup 1.0000x reference)
import functools

import jax
import jax.numpy as jnp
from jax import lax
from jax.experimental import pallas as pl
from jax.experimental.pallas import tpu as pltpu
from jax.experimental.pallas import tpu_sc as plsc

N = 10000
NP = 10240
D = 128
E = 320000
NC = 2
NS = 16
NW = NC * NS
EPT = E // NW
C = 80
NCH = EPT // C
RPT = NP // NS

_mesh = plsc.VectorSubcoreMesh(
    core_axis_name="c", subcore_axis_name="s", num_cores=NC, num_subcores=NS
)


_DEG_KERNEL_ARGS = dict(
    out_type=jax.ShapeDtypeStruct((NC, NP, D), jnp.float32),
    mesh=_mesh,
    scratch_types=[
        pltpu.VMEM((NCH, C), jnp.int32),
        pltpu.VMEM((2, C, D), jnp.float32),
        pltpu.SemaphoreType.DMA,
        pltpu.SemaphoreType.DMA,
        pltpu.SemaphoreType.DMA,
        pltpu.SemaphoreType.DMA,
        pltpu.VMEM_SHARED((NP, D), jnp.float32),
    ],
)


def _sc_degree_body(col_hbm, deg_hbm, idx_c, rows, sg0, sg1, ss0, ss1, acc):
    c = lax.axis_index("c")
    s = lax.axis_index("s")
    sg = (sg0, sg1)
    ss = (ss0, ss1)
    pltpu.async_copy(col_hbm.at[c, s], idx_c, ss[0])

    def _zero(i, _):
        for j in range(D // 16):
            z = jnp.zeros((16,), jnp.float32)
            rows[0, i, pl.ds(j * 16, 16)] = z
            rows[1, i, pl.ds(j * 16, 16)] = z
        return 0

    lax.fori_loop(0, C, _zero, 0)
    base = s * RPT
    for t in range(RPT // C):
        pltpu.async_copy(rows.at[t % 2], acc.at[pl.ds(base + t * C, C)], sg[t % 2])

    def _one(i, _):
        for j in range(D // 16):
            rows[0, i, pl.ds(j * 16, 16)] = jnp.full((16,), 1.0, jnp.float32)
        return 0

    for t in range(RPT // C):
        pltpu.make_async_copy(rows.at[t % 2], acc.at[pl.ds(base + t * C, C)], sg[t % 2]).wait()
    lax.fori_loop(0, C, _one, 0)
    pltpu.make_async_copy(col_hbm.at[c, s], idx_c, ss[0]).wait()
    plsc.subcore_barrier()

    def _pair(jj, _):
        for b in range(2):
            j = 2 * jj + b

            @pl.when(j >= 2)
            def _():
                pltpu.make_async_copy(rows.at[0], acc.at[idx_c.at[j - 2]], ss[b]).wait()

            pltpu.async_copy(rows.at[0], acc.at[idx_c.at[j]], ss[b], add=True)
        return 0

    lax.fori_loop(0, NCH // 2, _pair, 0)
    pltpu.make_async_copy(rows.at[0], acc.at[idx_c.at[NCH - 3]], ss[0]).wait()
    pltpu.async_copy(rows.at[0], acc.at[idx_c.at[NCH - 1]], ss[0], add=True)
    pltpu.make_async_copy(rows.at[0], acc.at[idx_c.at[NCH - 2]], ss[1]).wait()
    pltpu.make_async_copy(rows.at[0], acc.at[idx_c.at[NCH - 1]], ss[0]).wait()
    plsc.subcore_barrier()
    NT = RPT // C
    pltpu.async_copy(acc.at[pl.ds(base, C)], rows.at[0], sg[0])
    for t in range(NT):
        b = t % 2
        pltpu.make_async_copy(acc.at[pl.ds(base + t * C, C)], rows.at[b], sg[b]).wait()
        if t + 1 < NT:
            if t >= 1:
                pltpu.make_async_copy(
                    rows.at[1 - b], deg_hbm.at[c, pl.ds(base + (t - 1) * C, C)], ss[1 - b]
                ).wait()
            pltpu.async_copy(acc.at[pl.ds(base + (t + 1) * C, C)], rows.at[1 - b], sg[1 - b])
        pltpu.async_copy(rows.at[b], deg_hbm.at[c, pl.ds(base + t * C, C)], ss[b])
    pltpu.make_async_copy(rows.at[0], deg_hbm.at[c, pl.ds(base + (NT - 2) * C, C)], ss[0]).wait()
    pltpu.make_async_copy(rows.at[1], deg_hbm.at[c, pl.ds(base + (NT - 1) * C, C)], ss[1]).wait()


_sc_degree = pl.kernel(_sc_degree_body, **_DEG_KERNEL_ARGS)

_PROP_KERNEL_ARGS = dict(
    out_type=jax.ShapeDtypeStruct((NC, NP, D), jnp.float32),
    mesh=_mesh,
    scratch_types=[
        pltpu.VMEM((NCH, C), jnp.int32),
        pltpu.VMEM((C,), jnp.int32),
        pltpu.VMEM((C,), jnp.int32),
        pltpu.VMEM((2, C, D), jnp.float32),
        pltpu.SemaphoreType.DMA,
        pltpu.SemaphoreType.DMA,
        pltpu.SemaphoreType.DMA,
        pltpu.SemaphoreType.DMA,
        pltpu.SemaphoreType.DMA,
        pltpu.SemaphoreType.DMA,
        pltpu.VMEM_SHARED((NP, D), jnp.float32),
    ],
)


def _sc_propagate_body(g_hbm, row_hbm, col_hbm, u_hbm, idx_c, ir0, ir1, rows,
                       sg0, sg1, ss0, ss1, si0, si1, acc):
    c = lax.axis_index("c")
    s = lax.axis_index("s")
    ir = (ir0, ir1)
    sg = (sg0, sg1)
    ss = (ss0, ss1)
    si = (si0, si1)
    pltpu.async_copy(col_hbm.at[c, s], idx_c, ss[0])

    def _zero(i, _):
        for j in range(D // 16):
            z = jnp.zeros((16,), jnp.float32)
            rows[0, i, pl.ds(j * 16, 16)] = z
            rows[1, i, pl.ds(j * 16, 16)] = z
        return 0

    lax.fori_loop(0, C, _zero, 0)
    base = s * RPT
    for t in range(RPT // C):
        pltpu.async_copy(rows.at[t % 2], acc.at[pl.ds(base + t * C, C)], sg[t % 2])
    pltpu.async_copy(row_hbm.at[c, s, 0], ir[0], si[0])
    pltpu.async_copy(row_hbm.at[c, s, 1], ir[1], si[1])
    for t in range(RPT // C):
        pltpu.make_async_copy(rows.at[t % 2], acc.at[pl.ds(base + t * C, C)], sg[t % 2]).wait()
    pltpu.make_async_copy(col_hbm.at[c, s], idx_c, ss[0]).wait()
    pltpu.make_async_copy(row_hbm.at[c, s, 0], ir[0], si[0]).wait()
    pltpu.async_copy(g_hbm.at[ir[0]], rows.at[0], sg[0])
    plsc.subcore_barrier()

    def _pair(jj, _):
        for b in range(2):
            j = 2 * jj + b
            if b == 0:
                @pl.when(j >= 1)
                def _():
                    pltpu.make_async_copy(rows.at[1], acc.at[idx_c.at[j - 1]], ss[1]).wait()
            else:
                pltpu.make_async_copy(rows.at[0], acc.at[idx_c.at[j - 1]], ss[0]).wait()
            pltpu.make_async_copy(row_hbm.at[c, s, j + 1], ir[1 - b], si[1 - b]).wait()
            pltpu.async_copy(g_hbm.at[ir[1 - b]], rows.at[1 - b], sg[1 - b])
            pltpu.make_async_copy(g_hbm.at[ir[b]], rows.at[b], sg[b]).wait()
            pltpu.async_copy(rows.at[b], acc.at[idx_c.at[j]], ss[b], add=True)
            if b == 0:
                pltpu.async_copy(row_hbm.at[c, s, j + 2], ir[0], si[0])
            else:
                @pl.when(j + 2 < NCH)
                def _():
                    pltpu.async_copy(row_hbm.at[c, s, j + 2], ir[1], si[1])
        return 0

    lax.fori_loop(0, NCH // 2, _pair, 0)
    pltpu.make_async_copy(rows.at[1], acc.at[idx_c.at[NCH - 2]], ss[1]).wait()
    pltpu.make_async_copy(g_hbm.at[ir[0]], rows.at[0], sg[0]).wait()
    pltpu.sync_copy(rows.at[0], acc.at[idx_c.at[NCH - 1]], add=True)
    plsc.subcore_barrier()
    NT = RPT // C
    pltpu.async_copy(acc.at[pl.ds(base, C)], rows.at[0], sg[0])
    for t in range(NT):
        b = t % 2
        pltpu.make_async_copy(acc.at[pl.ds(base + t * C, C)], rows.at[b], sg[b]).wait()
        if t + 1 < NT:
            if t >= 1:
                pltpu.make_async_copy(
                    rows.at[1 - b], u_hbm.at[c, pl.ds(base + (t - 1) * C, C)], ss[1 - b]
                ).wait()
            pltpu.async_copy(acc.at[pl.ds(base + (t + 1) * C, C)], rows.at[1 - b], sg[1 - b])
        pltpu.async_copy(rows.at[b], u_hbm.at[c, pl.ds(base + t * C, C)], ss[b])
    pltpu.make_async_copy(rows.at[0], u_hbm.at[c, pl.ds(base + (NT - 2) * C, C)], ss[0]).wait()
    pltpu.make_async_copy(rows.at[1], u_hbm.at[c, pl.ds(base + (NT - 1) * C, C)], ss[1]).wait()


_sc_propagate = pl.kernel(_sc_propagate_body, **_PROP_KERNEL_ARGS)

_BN = 2000


def _dinv_of(deg):
    pos = deg > 0.0
    return jnp.where(pos, lax.rsqrt(jnp.where(pos, deg, 1.0)), 0.0)


def _tc_start_body(relu, h_ref, w_ref, b_ref, deg_ref, out_ref, g_ref):
    a = h_ref[...]
    if relu:
        a = jnp.maximum(a, 0.0)
    dinv = _dinv_of(deg_ref[...])
    out_ref[...] = jnp.dot(a, w_ref[...], preferred_element_type=jnp.float32) + b_ref[...]
    g_ref[...] = a * dinv


def _tc_start(h, w0, b, deg, relu):
    return pl.pallas_call(
        functools.partial(_tc_start_body, relu),
        grid=(N // _BN,),
        in_specs=[
            pl.BlockSpec((_BN, D), lambda i: (i, 0)),
            pl.BlockSpec((D, D), lambda i: (0, 0)),
            pl.BlockSpec((1, D), lambda i: (0, 0)),
            pl.BlockSpec((_BN, 1), lambda i: (i, 0)),
        ],
        out_specs=[
            pl.BlockSpec((_BN, D), lambda i: (i, 0)),
            pl.BlockSpec((_BN, D), lambda i: (i, 0)),
        ],
        out_shape=[
            jax.ShapeDtypeStruct((N, D), jnp.float32),
            jax.ShapeDtypeStruct((N, D), jnp.float32),
        ],
    )(h, w0, b, deg)


def _tc_hop_body(want_g, u0_ref, u1_ref, deg_ref, w_ref, oin_ref, out_ref, *g_ref):
    dinv = _dinv_of(deg_ref[...])
    h = (u0_ref[...] + u1_ref[...]) * dinv
    out_ref[...] = oin_ref[...] + jnp.dot(h, w_ref[...], preferred_element_type=jnp.float32)
    if want_g:
        g_ref[0][...] = h * dinv


def _tc_hop(u0, u1, deg, w, out_in, want_g):
    n_out = 2 if want_g else 1
    return pl.pallas_call(
        functools.partial(_tc_hop_body, want_g),
        grid=(N // _BN,),
        in_specs=[
            pl.BlockSpec((_BN, D), lambda i: (i, 0)),
            pl.BlockSpec((_BN, D), lambda i: (i, 0)),
            pl.BlockSpec((_BN, 1), lambda i: (i, 0)),
            pl.BlockSpec((D, D), lambda i: (0, 0)),
            pl.BlockSpec((_BN, D), lambda i: (i, 0)),
        ],
        out_specs=[pl.BlockSpec((_BN, D), lambda i: (i, 0))] * n_out,
        out_shape=[jax.ShapeDtypeStruct((N, D), jnp.float32)] * n_out,
        input_output_aliases={4: 0},
    )(u0, u1, deg, w, out_in)


def _tc_hop_start_body(u0_ref, u1_ref, deg_ref, w_ref, oin_ref, w0n_ref, bn_ref,
                       out_ref, g_ref):
    dinv = _dinv_of(deg_ref[...])
    h = (u0_ref[...] + u1_ref[...]) * dinv
    a = jnp.maximum(
        oin_ref[...] + jnp.dot(h, w_ref[...], preferred_element_type=jnp.float32), 0.0
    )
    out_ref[...] = jnp.dot(a, w0n_ref[...], preferred_element_type=jnp.float32) + bn_ref[...]
    g_ref[...] = a * dinv


def _tc_hop_start(u0, u1, deg, w, out_in, w0n, bn):
    return pl.pallas_call(
        _tc_hop_start_body,
        grid=(N // _BN,),
        in_specs=[
            pl.BlockSpec((_BN, D), lambda i: (i, 0)),
            pl.BlockSpec((_BN, D), lambda i: (i, 0)),
            pl.BlockSpec((_BN, 1), lambda i: (i, 0)),
            pl.BlockSpec((D, D), lambda i: (0, 0)),
            pl.BlockSpec((_BN, D), lambda i: (i, 0)),
            pl.BlockSpec((D, D), lambda i: (0, 0)),
            pl.BlockSpec((1, D), lambda i: (0, 0)),
        ],
        out_specs=[pl.BlockSpec((_BN, D), lambda i: (i, 0))] * 2,
        out_shape=[jax.ShapeDtypeStruct((N, D), jnp.float32)] * 2,
    )(u0, u1, deg, w, out_in, w0n, bn)


def kernel(x, edge_index, W0, b0, W1, b1, W2, b2):
    row = edge_index[0].astype(jnp.int32).reshape(NC, NS, NCH, C)
    col = edge_index[1].astype(jnp.int32).reshape(NC, NS, NCH, C)
    degp = _sc_degree(col)
    deg = degp[0, :, 0:1] + degp[1, :, 0:1]

    Ws = (W0, W1, W2)
    bs = (b0, b1, b2)
    out_l, g = _tc_start(x, W0[0], b0.reshape(1, D), deg, False)
    for l in range(3):
        W = Ws[l]
        for k in range(1, 4):
            u = _sc_propagate(g, row, col)
            if k < 3:
                out_l, g = _tc_hop(u[0], u[1], deg, W[k], out_l, True)
            elif l < 2:
                out_l, g = _tc_hop_start(
                    u[0], u[1], deg, W[3], out_l, Ws[l + 1][0], bs[l + 1].reshape(1, D)
                )
            else:
                out_l = _tc_hop(u[0], u[1], deg, W[3], out_l, False)[0]
    return out_l


# speedup vs baseline: 1.0033x; 1.0033x over previous
import functools

import jax
import jax.numpy as jnp
from jax import lax
from jax.experimental import pallas as pl
from jax.experimental.pallas import tpu as pltpu
from jax.experimental.pallas import tpu_sc as plsc

N = 10000
NP = 10240
D = 128
E = 320000
NC = 2
NS = 16
NW = NC * NS
EPT = E // NW
C = 80
NCH = EPT // C
RPT = NP // NS

_mesh = plsc.VectorSubcoreMesh(
    core_axis_name="c", subcore_axis_name="s", num_cores=NC, num_subcores=NS
)


_DEG_KERNEL_ARGS = dict(
    out_type=jax.ShapeDtypeStruct((NC, NP, D), jnp.float32),
    mesh=_mesh,
    scratch_types=[
        pltpu.VMEM((NCH, C), jnp.int32),
        pltpu.VMEM((2, C, D), jnp.float32),
        pltpu.SemaphoreType.DMA,
        pltpu.SemaphoreType.DMA,
        pltpu.SemaphoreType.DMA,
        pltpu.SemaphoreType.DMA,
        pltpu.VMEM_SHARED((NP, D), jnp.float32),
    ],
)


def _sc_degree_body(col_hbm, deg_hbm, idx_c, rows, sg0, sg1, ss0, ss1, acc):
    c = lax.axis_index("c")
    s = lax.axis_index("s")
    sg = (sg0, sg1)
    ss = (ss0, ss1)
    pltpu.async_copy(col_hbm.at[c, s], idx_c, ss[0])

    def _zero(i, _):
        for j in range(D // 16):
            rows[1, i, pl.ds(j * 16, 16)] = jnp.zeros((16,), jnp.float32)
        return 0

    lax.fori_loop(0, C, _zero, 0)
    base = s * RPT
    for t in range(RPT // C):
        pltpu.async_copy(rows.at[1], acc.at[pl.ds(base + t * C, C)], sg[0])

    def _one(i, _):
        for j in range(D // 16):
            rows[0, i, pl.ds(j * 16, 16)] = jnp.full((16,), 1.0, jnp.float32)
        return 0

    lax.fori_loop(0, C, _one, 0)
    for t in range(RPT // C):
        pltpu.make_async_copy(rows.at[1], acc.at[pl.ds(base + t * C, C)], sg[0]).wait()
    pltpu.make_async_copy(col_hbm.at[c, s], idx_c, ss[0]).wait()
    plsc.subcore_barrier()

    def _pair(jj, _):
        for b in range(2):
            j = 2 * jj + b

            @pl.when(j >= 2)
            def _():
                pltpu.make_async_copy(rows.at[0], acc.at[idx_c.at[j - 2]], ss[b]).wait()

            pltpu.async_copy(rows.at[0], acc.at[idx_c.at[j]], ss[b], add=True)
        return 0

    lax.fori_loop(0, NCH // 2, _pair, 0)
    pltpu.make_async_copy(rows.at[0], acc.at[idx_c.at[NCH - 3]], ss[0]).wait()
    pltpu.async_copy(rows.at[0], acc.at[idx_c.at[NCH - 1]], ss[0], add=True)
    pltpu.make_async_copy(rows.at[0], acc.at[idx_c.at[NCH - 2]], ss[1]).wait()
    pltpu.make_async_copy(rows.at[0], acc.at[idx_c.at[NCH - 1]], ss[0]).wait()
    plsc.subcore_barrier()
    NT = RPT // C
    pltpu.async_copy(acc.at[pl.ds(base, C)], rows.at[0], sg[0])
    for t in range(NT):
        b = t % 2
        pltpu.make_async_copy(acc.at[pl.ds(base + t * C, C)], rows.at[b], sg[b]).wait()
        if t + 1 < NT:
            if t >= 1:
                pltpu.make_async_copy(
                    rows.at[1 - b], deg_hbm.at[c, pl.ds(base + (t - 1) * C, C)], ss[1 - b]
                ).wait()
            pltpu.async_copy(acc.at[pl.ds(base + (t + 1) * C, C)], rows.at[1 - b], sg[1 - b])
        pltpu.async_copy(rows.at[b], deg_hbm.at[c, pl.ds(base + t * C, C)], ss[b])
    pltpu.make_async_copy(rows.at[0], deg_hbm.at[c, pl.ds(base + (NT - 2) * C, C)], ss[0]).wait()
    pltpu.make_async_copy(rows.at[1], deg_hbm.at[c, pl.ds(base + (NT - 1) * C, C)], ss[1]).wait()


_sc_degree = pl.kernel(_sc_degree_body, **_DEG_KERNEL_ARGS)

_PROP_KERNEL_ARGS = dict(
    out_type=jax.ShapeDtypeStruct((NC, NP, D), jnp.float32),
    mesh=_mesh,
    scratch_types=[
        pltpu.VMEM((NCH, C), jnp.int32),
        pltpu.VMEM((C,), jnp.int32),
        pltpu.VMEM((C,), jnp.int32),
        pltpu.VMEM((2, C, D), jnp.float32),
        pltpu.SemaphoreType.DMA,
        pltpu.SemaphoreType.DMA,
        pltpu.SemaphoreType.DMA,
        pltpu.SemaphoreType.DMA,
        pltpu.SemaphoreType.DMA,
        pltpu.SemaphoreType.DMA,
        pltpu.VMEM_SHARED((NP, D), jnp.float32),
    ],
)


def _sc_propagate_body(g_hbm, row_hbm, col_hbm, u_hbm, idx_c, ir0, ir1, rows,
                       sg0, sg1, ss0, ss1, si0, si1, acc):
    c = lax.axis_index("c")
    s = lax.axis_index("s")
    ir = (ir0, ir1)
    sg = (sg0, sg1)
    ss = (ss0, ss1)
    si = (si0, si1)
    pltpu.async_copy(col_hbm.at[c, s], idx_c, ss[0])

    def _zero(i, _):
        for j in range(D // 16):
            rows[0, i, pl.ds(j * 16, 16)] = jnp.zeros((16,), jnp.float32)
        return 0

    lax.fori_loop(0, C, _zero, 0)
    base = s * RPT
    for t in range(RPT // C):
        pltpu.async_copy(rows.at[0], acc.at[pl.ds(base + t * C, C)], sg[0])
    pltpu.async_copy(row_hbm.at[c, s, 0], ir[0], si[0])
    pltpu.async_copy(row_hbm.at[c, s, 1], ir[1], si[1])
    for t in range(RPT // C):
        pltpu.make_async_copy(rows.at[0], acc.at[pl.ds(base + t * C, C)], sg[0]).wait()
    pltpu.make_async_copy(col_hbm.at[c, s], idx_c, ss[0]).wait()
    pltpu.make_async_copy(row_hbm.at[c, s, 0], ir[0], si[0]).wait()
    pltpu.async_copy(g_hbm.at[ir[0]], rows.at[0], sg[0])
    plsc.subcore_barrier()

    def _pair(jj, _):
        for b in range(2):
            j = 2 * jj + b
            if b == 0:
                @pl.when(j >= 1)
                def _():
                    pltpu.make_async_copy(rows.at[1], acc.at[idx_c.at[j - 1]], ss[1]).wait()
            else:
                pltpu.make_async_copy(rows.at[0], acc.at[idx_c.at[j - 1]], ss[0]).wait()
            pltpu.make_async_copy(row_hbm.at[c, s, j + 1], ir[1 - b], si[1 - b]).wait()
            pltpu.async_copy(g_hbm.at[ir[1 - b]], rows.at[1 - b], sg[1 - b])
            pltpu.make_async_copy(g_hbm.at[ir[b]], rows.at[b], sg[b]).wait()
            if b == 0:
                pltpu.async_copy(row_hbm.at[c, s, j + 2], ir[0], si[0])
            else:
                @pl.when(j + 2 < NCH)
                def _():
                    pltpu.async_copy(row_hbm.at[c, s, j + 2], ir[1], si[1])
            pltpu.async_copy(rows.at[b], acc.at[idx_c.at[j]], ss[b], add=True)
        return 0

    lax.fori_loop(0, NCH // 2, _pair, 0)
    pltpu.make_async_copy(rows.at[1], acc.at[idx_c.at[NCH - 2]], ss[1]).wait()
    pltpu.make_async_copy(g_hbm.at[ir[0]], rows.at[0], sg[0]).wait()
    pltpu.sync_copy(rows.at[0], acc.at[idx_c.at[NCH - 1]], add=True)
    plsc.subcore_barrier()
    NT = RPT // C
    pltpu.async_copy(acc.at[pl.ds(base, C)], rows.at[0], sg[0])
    for t in range(NT):
        b = t % 2
        pltpu.make_async_copy(acc.at[pl.ds(base + t * C, C)], rows.at[b], sg[b]).wait()
        if t + 1 < NT:
            if t >= 1:
                pltpu.make_async_copy(
                    rows.at[1 - b], u_hbm.at[c, pl.ds(base + (t - 1) * C, C)], ss[1 - b]
                ).wait()
            pltpu.async_copy(acc.at[pl.ds(base + (t + 1) * C, C)], rows.at[1 - b], sg[1 - b])
        pltpu.async_copy(rows.at[b], u_hbm.at[c, pl.ds(base + t * C, C)], ss[b])
    pltpu.make_async_copy(rows.at[0], u_hbm.at[c, pl.ds(base + (NT - 2) * C, C)], ss[0]).wait()
    pltpu.make_async_copy(rows.at[1], u_hbm.at[c, pl.ds(base + (NT - 1) * C, C)], ss[1]).wait()


_sc_propagate = pl.kernel(_sc_propagate_body, **_PROP_KERNEL_ARGS)

_BN = 2000


def _dinv_of(deg):
    pos = deg > 0.0
    return jnp.where(pos, lax.rsqrt(jnp.where(pos, deg, 1.0)), 0.0)


def _tc_start_body(relu, h_ref, w_ref, b_ref, deg_ref, out_ref, g_ref):
    a = h_ref[...]
    if relu:
        a = jnp.maximum(a, 0.0)
    dinv = _dinv_of(deg_ref[...])
    out_ref[...] = jnp.dot(a, w_ref[...], preferred_element_type=jnp.float32) + b_ref[...]
    g_ref[...] = a * dinv


def _tc_start(h, w0, b, deg, relu):
    return pl.pallas_call(
        functools.partial(_tc_start_body, relu),
        grid=(N // _BN,),
        in_specs=[
            pl.BlockSpec((_BN, D), lambda i: (i, 0)),
            pl.BlockSpec((D, D), lambda i: (0, 0)),
            pl.BlockSpec((1, D), lambda i: (0, 0)),
            pl.BlockSpec((_BN, 1), lambda i: (i, 0)),
        ],
        out_specs=[
            pl.BlockSpec((_BN, D), lambda i: (i, 0)),
            pl.BlockSpec((_BN, D), lambda i: (i, 0)),
        ],
        out_shape=[
            jax.ShapeDtypeStruct((N, D), jnp.float32),
            jax.ShapeDtypeStruct((N, D), jnp.float32),
        ],
    )(h, w0, b, deg)


def _tc_hop_body(want_g, u0_ref, u1_ref, deg_ref, w_ref, oin_ref, out_ref, *g_ref):
    dinv = _dinv_of(deg_ref[...])
    h = (u0_ref[...] + u1_ref[...]) * dinv
    out_ref[...] = oin_ref[...] + jnp.dot(h, w_ref[...], preferred_element_type=jnp.float32)
    if want_g:
        g_ref[0][...] = h * dinv


def _tc_hop(u0, u1, deg, w, out_in, want_g):
    n_out = 2 if want_g else 1
    return pl.pallas_call(
        functools.partial(_tc_hop_body, want_g),
        grid=(N // _BN,),
        in_specs=[
            pl.BlockSpec((_BN, D), lambda i: (i, 0)),
            pl.BlockSpec((_BN, D), lambda i: (i, 0)),
            pl.BlockSpec((_BN, 1), lambda i: (i, 0)),
            pl.BlockSpec((D, D), lambda i: (0, 0)),
            pl.BlockSpec((_BN, D), lambda i: (i, 0)),
        ],
        out_specs=[pl.BlockSpec((_BN, D), lambda i: (i, 0))] * n_out,
        out_shape=[jax.ShapeDtypeStruct((N, D), jnp.float32)] * n_out,
        input_output_aliases={4: 0},
    )(u0, u1, deg, w, out_in)


def _tc_hop_start_body(u0_ref, u1_ref, deg_ref, w_ref, oin_ref, w0n_ref, bn_ref,
                       out_ref, g_ref):
    dinv = _dinv_of(deg_ref[...])
    h = (u0_ref[...] + u1_ref[...]) * dinv
    a = jnp.maximum(
        oin_ref[...] + jnp.dot(h, w_ref[...], preferred_element_type=jnp.float32), 0.0
    )
    out_ref[...] = jnp.dot(a, w0n_ref[...], preferred_element_type=jnp.float32) + bn_ref[...]
    g_ref[...] = a * dinv


def _tc_hop_start(u0, u1, deg, w, out_in, w0n, bn):
    return pl.pallas_call(
        _tc_hop_start_body,
        grid=(N // _BN,),
        in_specs=[
            pl.BlockSpec((_BN, D), lambda i: (i, 0)),
            pl.BlockSpec((_BN, D), lambda i: (i, 0)),
            pl.BlockSpec((_BN, 1), lambda i: (i, 0)),
            pl.BlockSpec((D, D), lambda i: (0, 0)),
            pl.BlockSpec((_BN, D), lambda i: (i, 0)),
            pl.BlockSpec((D, D), lambda i: (0, 0)),
            pl.BlockSpec((1, D), lambda i: (0, 0)),
        ],
        out_specs=[pl.BlockSpec((_BN, D), lambda i: (i, 0))] * 2,
        out_shape=[jax.ShapeDtypeStruct((N, D), jnp.float32)] * 2,
    )(u0, u1, deg, w, out_in, w0n, bn)


def kernel(x, edge_index, W0, b0, W1, b1, W2, b2):
    row = edge_index[0].astype(jnp.int32).reshape(NC, NS, NCH, C)
    col = edge_index[1].astype(jnp.int32).reshape(NC, NS, NCH, C)
    degp = _sc_degree(col)
    deg = degp[0, :, 0:1] + degp[1, :, 0:1]

    Ws = (W0, W1, W2)
    bs = (b0, b1, b2)
    out_l, g = _tc_start(x, W0[0], b0.reshape(1, D), deg, False)
    for l in range(3):
        W = Ws[l]
        for k in range(1, 4):
            u = _sc_propagate(g, row, col)
            if k < 3:
                out_l, g = _tc_hop(u[0], u[1], deg, W[k], out_l, True)
            elif l < 2:
                out_l, g = _tc_hop_start(
                    u[0], u[1], deg, W[3], out_l, Ws[l + 1][0], bs[l + 1].reshape(1, D)
                )
            else:
                out_l = _tc_hop(u[0], u[1], deg, W[3], out_l, False)[0]
    return out_l


# speedup vs baseline: 1.0052x; 1.0019x over previous
import functools

import jax
import jax.numpy as jnp
from jax import lax
from jax.experimental import pallas as pl
from jax.experimental.pallas import tpu as pltpu
from jax.experimental.pallas import tpu_sc as plsc

N = 10000
NP = 10240
D = 128
E = 320000
NC = 2
NS = 16
NW = NC * NS
EPT = E // NW
C = 80
NCH = EPT // C
RPT = NP // NS

_mesh = plsc.VectorSubcoreMesh(
    core_axis_name="c", subcore_axis_name="s", num_cores=NC, num_subcores=NS
)


_DEG_KERNEL_ARGS = dict(
    out_type=jax.ShapeDtypeStruct((NC, NP, D), jnp.float32),
    mesh=_mesh,
    scratch_types=[
        pltpu.VMEM((NCH, C), jnp.int32),
        pltpu.VMEM((2, C, D), jnp.float32),
        pltpu.SemaphoreType.DMA,
        pltpu.SemaphoreType.DMA,
        pltpu.SemaphoreType.DMA,
        pltpu.SemaphoreType.DMA,
        pltpu.VMEM_SHARED((NP, D), jnp.float32),
    ],
)


def _sc_degree_body(col_hbm, deg_hbm, idx_c, rows, sg0, sg1, ss0, ss1, acc):
    c = lax.axis_index("c")
    s = lax.axis_index("s")
    sg = (sg0, sg1)
    ss = (ss0, ss1)
    pltpu.async_copy(col_hbm.at[c, s], idx_c, ss[0])

    def _zero(i, _):
        for j in range(D // 16):
            rows[1, i, pl.ds(j * 16, 16)] = jnp.zeros((16,), jnp.float32)
        return 0

    lax.fori_loop(0, C, _zero, 0)
    base = s * RPT
    for t in range(RPT // C):
        pltpu.async_copy(rows.at[1], acc.at[pl.ds(base + t * C, C)], sg[0])

    def _one(i, _):
        for j in range(D // 16):
            rows[0, i, pl.ds(j * 16, 16)] = jnp.full((16,), 1.0, jnp.float32)
        return 0

    lax.fori_loop(0, C, _one, 0)
    for t in range(RPT // C):
        pltpu.make_async_copy(rows.at[1], acc.at[pl.ds(base + t * C, C)], sg[0]).wait()
    pltpu.make_async_copy(col_hbm.at[c, s], idx_c, ss[0]).wait()
    plsc.subcore_barrier()

    def _pair(jj, _):
        for b in range(2):
            j = 2 * jj + b

            @pl.when(j >= 2)
            def _():
                pltpu.make_async_copy(rows.at[0], acc.at[idx_c.at[j - 2]], ss[b]).wait()

            pltpu.async_copy(rows.at[0], acc.at[idx_c.at[j]], ss[b], add=True)
        return 0

    lax.fori_loop(0, NCH // 2, _pair, 0)
    pltpu.make_async_copy(rows.at[0], acc.at[idx_c.at[NCH - 3]], ss[0]).wait()
    pltpu.async_copy(rows.at[0], acc.at[idx_c.at[NCH - 1]], ss[0], add=True)
    pltpu.make_async_copy(rows.at[0], acc.at[idx_c.at[NCH - 2]], ss[1]).wait()
    pltpu.make_async_copy(rows.at[0], acc.at[idx_c.at[NCH - 1]], ss[0]).wait()
    plsc.subcore_barrier()
    NT = RPT // C
    pltpu.async_copy(acc.at[pl.ds(base, C)], rows.at[0], sg[0])
    for t in range(NT):
        b = t % 2
        pltpu.make_async_copy(acc.at[pl.ds(base + t * C, C)], rows.at[b], sg[b]).wait()
        if t + 1 < NT:
            if t >= 1:
                pltpu.make_async_copy(
                    rows.at[1 - b], deg_hbm.at[c, pl.ds(base + (t - 1) * C, C)], ss[1 - b]
                ).wait()
            pltpu.async_copy(acc.at[pl.ds(base + (t + 1) * C, C)], rows.at[1 - b], sg[1 - b])
        pltpu.async_copy(rows.at[b], deg_hbm.at[c, pl.ds(base + t * C, C)], ss[b])
    pltpu.make_async_copy(rows.at[0], deg_hbm.at[c, pl.ds(base + (NT - 2) * C, C)], ss[0]).wait()
    pltpu.make_async_copy(rows.at[1], deg_hbm.at[c, pl.ds(base + (NT - 1) * C, C)], ss[1]).wait()


_sc_degree = pl.kernel(_sc_degree_body, **_DEG_KERNEL_ARGS)

_PROP_KERNEL_ARGS = dict(
    out_type=jax.ShapeDtypeStruct((NC, NP, D), jnp.float32),
    mesh=_mesh,
    scratch_types=[
        pltpu.VMEM((NCH, C), jnp.int32),
        pltpu.VMEM((C,), jnp.int32),
        pltpu.VMEM((C,), jnp.int32),
        pltpu.VMEM((2, C, D), jnp.float32),
        pltpu.SemaphoreType.DMA,
        pltpu.SemaphoreType.DMA,
        pltpu.SemaphoreType.DMA,
        pltpu.SemaphoreType.DMA,
        pltpu.SemaphoreType.DMA,
        pltpu.SemaphoreType.DMA,
        pltpu.VMEM_SHARED((NP, D), jnp.float32),
    ],
)


def _sc_propagate_body(g_hbm, row_hbm, col_hbm, u_hbm, idx_c, ir0, ir1, rows,
                       sg0, sg1, ss0, ss1, si0, si1, acc):
    c = lax.axis_index("c")
    s = lax.axis_index("s")
    ir = (ir0, ir1)
    sg = (sg0, sg1)
    ss = (ss0, ss1)
    si = (si0, si1)
    pltpu.async_copy(col_hbm.at[c, s], idx_c, ss[0])

    def _zero(i, _):
        for j in range(D // 16):
            rows[0, i, pl.ds(j * 16, 16)] = jnp.zeros((16,), jnp.float32)
        return 0

    lax.fori_loop(0, C, _zero, 0)
    base = s * RPT
    for t in range(RPT // C):
        pltpu.async_copy(rows.at[0], acc.at[pl.ds(base + t * C, C)], sg[0])
    pltpu.async_copy(row_hbm.at[c, s, 0], ir[0], si[0])
    pltpu.async_copy(row_hbm.at[c, s, 1], ir[1], si[1])
    for t in range(RPT // C):
        pltpu.make_async_copy(rows.at[0], acc.at[pl.ds(base + t * C, C)], sg[0]).wait()
    pltpu.make_async_copy(col_hbm.at[c, s], idx_c, ss[0]).wait()
    pltpu.make_async_copy(row_hbm.at[c, s, 0], ir[0], si[0]).wait()
    pltpu.async_copy(g_hbm.at[ir[0]], rows.at[0], sg[0])
    plsc.subcore_barrier()

    def _pair(jj, _):
        for b in range(2):
            j = 2 * jj + b
            if b == 0:
                @pl.when(j >= 1)
                def _():
                    pltpu.make_async_copy(rows.at[1], acc.at[idx_c.at[j - 1]], ss[1]).wait()
            else:
                pltpu.make_async_copy(rows.at[0], acc.at[idx_c.at[j - 1]], ss[0]).wait()
            pltpu.make_async_copy(row_hbm.at[c, s, j + 1], ir[1 - b], si[1 - b]).wait()
            pltpu.async_copy(g_hbm.at[ir[1 - b]], rows.at[1 - b], sg[1 - b])
            pltpu.make_async_copy(g_hbm.at[ir[b]], rows.at[b], sg[b]).wait()
            if b == 0:
                pltpu.async_copy(row_hbm.at[c, s, j + 2], ir[0], si[0])
            else:
                @pl.when(j + 2 < NCH)
                def _():
                    pltpu.async_copy(row_hbm.at[c, s, j + 2], ir[1], si[1])
            pltpu.async_copy(rows.at[b], acc.at[idx_c.at[j]], ss[b], add=True)
        return 0

    lax.fori_loop(0, NCH // 2, _pair, 0)
    pltpu.make_async_copy(rows.at[1], acc.at[idx_c.at[NCH - 2]], ss[1]).wait()
    pltpu.make_async_copy(g_hbm.at[ir[0]], rows.at[0], sg[0]).wait()
    pltpu.sync_copy(rows.at[0], acc.at[idx_c.at[NCH - 1]], add=True)
    plsc.subcore_barrier()
    NT = RPT // C
    pltpu.async_copy(acc.at[pl.ds(base, C)], rows.at[0], sg[0])
    for t in range(NT):
        b = t % 2
        pltpu.make_async_copy(acc.at[pl.ds(base + t * C, C)], rows.at[b], sg[b]).wait()
        if t + 1 < NT:
            if t >= 1:
                pltpu.make_async_copy(
                    rows.at[1 - b], u_hbm.at[c, pl.ds(base + (t - 1) * C, C)], ss[1 - b]
                ).wait()
            pltpu.async_copy(acc.at[pl.ds(base + (t + 1) * C, C)], rows.at[1 - b], sg[1 - b])
        pltpu.async_copy(rows.at[b], u_hbm.at[c, pl.ds(base + t * C, C)], ss[b])
    pltpu.make_async_copy(rows.at[0], u_hbm.at[c, pl.ds(base + (NT - 2) * C, C)], ss[0]).wait()
    pltpu.make_async_copy(rows.at[1], u_hbm.at[c, pl.ds(base + (NT - 1) * C, C)], ss[1]).wait()


_sc_propagate = pl.kernel(_sc_propagate_body, **_PROP_KERNEL_ARGS)

_BN = 2000


def _dinv_of(deg):
    pos = deg > 0.0
    return jnp.where(pos, lax.rsqrt(jnp.where(pos, deg, 1.0)), 0.0)


def _tc_start_body(relu, h_ref, w_ref, b_ref, deg_ref, out_ref, g_ref):
    a = h_ref[...]
    if relu:
        a = jnp.maximum(a, 0.0)
    dinv = _dinv_of(deg_ref[...])
    out_ref[...] = jnp.dot(a, w_ref[...], preferred_element_type=jnp.float32) + b_ref[...]
    g_ref[...] = a * dinv


def _tc_start(h, w0, b, deg, relu):
    return pl.pallas_call(
        functools.partial(_tc_start_body, relu),
        grid=(N // _BN,),
        in_specs=[
            pl.BlockSpec((_BN, D), lambda i: (i, 0)),
            pl.BlockSpec((D, D), lambda i: (0, 0)),
            pl.BlockSpec((1, D), lambda i: (0, 0)),
            pl.BlockSpec((_BN, 1), lambda i: (i, 0)),
        ],
        out_specs=[
            pl.BlockSpec((_BN, D), lambda i: (i, 0)),
            pl.BlockSpec((_BN, D), lambda i: (i, 0)),
        ],
        out_shape=[
            jax.ShapeDtypeStruct((N, D), jnp.float32),
            jax.ShapeDtypeStruct((N, D), jnp.float32),
        ],
    )(h, w0, b, deg)


def _tc_hop_body(want_g, u0_ref, u1_ref, deg_ref, w_ref, oin_ref, out_ref, *g_ref):
    dinv = _dinv_of(deg_ref[...])
    h = (u0_ref[...] + u1_ref[...]) * dinv
    out_ref[...] = oin_ref[...] + jnp.dot(h, w_ref[...], preferred_element_type=jnp.float32)
    if want_g:
        g_ref[0][...] = h * dinv


def _tc_hop(u0, u1, deg, w, out_in, want_g):
    n_out = 2 if want_g else 1
    return pl.pallas_call(
        functools.partial(_tc_hop_body, want_g),
        grid=(N // _BN,),
        in_specs=[
            pl.BlockSpec((_BN, D), lambda i: (i, 0)),
            pl.BlockSpec((_BN, D), lambda i: (i, 0)),
            pl.BlockSpec((_BN, 1), lambda i: (i, 0)),
            pl.BlockSpec((D, D), lambda i: (0, 0)),
            pl.BlockSpec((_BN, D), lambda i: (i, 0)),
        ],
        out_specs=[pl.BlockSpec((_BN, D), lambda i: (i, 0))] * n_out,
        out_shape=[jax.ShapeDtypeStruct((N, D), jnp.float32)] * n_out,
        input_output_aliases={4: 0},
    )(u0, u1, deg, w, out_in)


def _tc_hop_start_body(u0_ref, u1_ref, deg_ref, w_ref, oin_ref, w0n_ref, bn_ref,
                       out_ref, g_ref):
    dinv = _dinv_of(deg_ref[...])
    h = (u0_ref[...] + u1_ref[...]) * dinv
    a = jnp.maximum(
        oin_ref[...] + jnp.dot(h, w_ref[...], preferred_element_type=jnp.float32), 0.0
    )
    out_ref[...] = jnp.dot(a, w0n_ref[...], preferred_element_type=jnp.float32) + bn_ref[...]
    g_ref[...] = a * dinv


def _tc_hop_start(u0, u1, deg, w, out_in, w0n, bn):
    return pl.pallas_call(
        _tc_hop_start_body,
        grid=(N // _BN,),
        in_specs=[
            pl.BlockSpec((_BN, D), lambda i: (i, 0)),
            pl.BlockSpec((_BN, D), lambda i: (i, 0)),
            pl.BlockSpec((_BN, 1), lambda i: (i, 0)),
            pl.BlockSpec((D, D), lambda i: (0, 0)),
            pl.BlockSpec((_BN, D), lambda i: (i, 0)),
            pl.BlockSpec((D, D), lambda i: (0, 0)),
            pl.BlockSpec((1, D), lambda i: (0, 0)),
        ],
        out_specs=[pl.BlockSpec((_BN, D), lambda i: (i, 0))] * 2,
        out_shape=[jax.ShapeDtypeStruct((N, D), jnp.float32)] * 2,
        input_output_aliases={4: 0},
    )(u0, u1, deg, w, out_in, w0n, bn)


def kernel(x, edge_index, W0, b0, W1, b1, W2, b2):
    row = edge_index[0].astype(jnp.int32).reshape(NC, NS, NCH, C)
    col = edge_index[1].astype(jnp.int32).reshape(NC, NS, NCH, C)
    degp = _sc_degree(col)
    deg = degp[0, :, 0:1] + degp[1, :, 0:1]

    Ws = (W0, W1, W2)
    bs = (b0, b1, b2)
    out_l, g = _tc_start(x, W0[0], b0.reshape(1, D), deg, False)
    for l in range(3):
        W = Ws[l]
        for k in range(1, 4):
            u = _sc_propagate(g, row, col)
            if k < 3:
                out_l, g = _tc_hop(u[0], u[1], deg, W[k], out_l, True)
            elif l < 2:
                out_l, g = _tc_hop_start(
                    u[0], u[1], deg, W[3], out_l, Ws[l + 1][0], bs[l + 1].reshape(1, D)
                )
            else:
                out_l = _tc_hop(u[0], u[1], deg, W[3], out_l, False)[0]
    return out_l
